# Initial kernel scaffold; baseline (speedup 1.0000x reference)
import jax
import jax.numpy as jnp
from jax.experimental import pallas as pl


def kernel(x, edge_index, edge_attr, batch, Wx1, We1, Wo1, bo1, eps1, Wx2, We2, Wo2, bo2, eps2, Wx3, We3, Wo3, bo3, eps3, lin1_W, lin1_b, lin2_W, lin2_b):
    raise NotImplementedError("write your pallas kernel here")


# speedup vs baseline: 2.9958x; 2.9958x over previous
import functools

import jax
import jax.numpy as jnp
from jax import lax
from jax.experimental import pallas as pl
from jax.experimental.pallas import tpu as pltpu
from jax.experimental.pallas import tpu_sc as plsc

N = 10000
E = 320000
D = 128
ED = 16
H = 128
NG = 64
OUT = 10

_NC = 2
_NS = 16
_NW = _NC * _NS
_L = 16
_C = 128
_NCHUNK = E // _C
_ITERS = (_NCHUNK + _NW - 1) // _NW
_RC = 80
_NZCH = N // _RC


def _mm_body(a_ref, w_ref, o_ref):
    o_ref[...] = jnp.dot(a_ref[...], w_ref[...],
                         preferred_element_type=jnp.float32)


def _mm(a, w, blk):
    m, k = a.shape
    n = w.shape[1]
    grid = m // blk
    return pl.pallas_call(
        _mm_body,
        grid=(grid,),
        in_specs=[
            pl.BlockSpec((blk, k), lambda i: (i, 0)),
            pl.BlockSpec((k, n), lambda i: (0, 0)),
        ],
        out_specs=pl.BlockSpec((blk, n), lambda i: (i, 0)),
        out_shape=jax.ShapeDtypeStruct((m, n), jnp.float32),
    )(a, w)


@functools.lru_cache(maxsize=None)
def _sc_edge_kernel():
    return functools.partial(
        pl.kernel,
        out_type=jax.ShapeDtypeStruct((_NC, N, H), jnp.float32),
        mesh=plsc.VectorSubcoreMesh(core_axis_name="c", subcore_axis_name="s",
                                    num_cores=_NC, num_subcores=_NS),
        scratch_types=[
            pltpu.VMEM((_C,), jnp.int32),
            pltpu.VMEM((_C,), jnp.int32),
            pltpu.VMEM((_C, H), jnp.float32),
            pltpu.VMEM((_C, H), jnp.float32),
            pltpu.VMEM_SHARED((N, H), jnp.float32),
            pltpu.SemaphoreType.DMA,
        ],
    )(_sc_edge_body)


def _sc_edge(hs, e, ei):
    return _sc_edge_kernel()(hs, e, ei)


def _sc_edge_body(hs_hbm, e_hbm, ei_hbm, out_hbm, src_v, dst_v, ebuf, rows,
                  agg, sem):
    c = lax.axis_index("c")
    s = lax.axis_index("s")
    wid = s * _NC + c

    zv = jnp.zeros((_L,), jnp.float32)

    @pl.loop(0, _RC)
    def _zero_rows(r):
        for j in range(H // _L):
            rows[r, pl.ds(j * _L, _L)] = zv

    for j in range((_NZCH + _NS - 1) // _NS):
        t = j * _NS + s

        @pl.when(t < _NZCH)
        def _():
            pltpu.sync_copy(rows.at[pl.ds(0, _RC)],
                            agg.at[pl.ds(t * _RC, _RC)])
    plsc.subcore_barrier()

    @pl.loop(0, _ITERS)
    def _chunk_loop(i):
        chunk = i * _NW + wid

        @pl.when(chunk < _NCHUNK)
        def _():
            base = chunk * _C
            pltpu.sync_copy(ei_hbm.at[0, pl.ds(base, _C)], src_v)
            pltpu.sync_copy(ei_hbm.at[1, pl.ds(base, _C)], dst_v)
            pltpu.sync_copy(e_hbm.at[pl.ds(base, _C), :], ebuf)
            pltpu.async_copy(hs_hbm.at[src_v], rows, sem).wait()

            @pl.loop(0, _C)
            def _relu_rows(r):
                for j in range(H // _L):
                    sl = pl.ds(j * _L, _L)
                    rows[r, sl] = jnp.maximum(rows[r, sl] + ebuf[r, sl], 0.0)

            pltpu.sync_copy(rows, agg.at[dst_v], add=True)

    plsc.subcore_barrier()
    for j in range((_NZCH + _NS - 1) // _NS):
        t = j * _NS + s

        @pl.when(t < _NZCH)
        def _():
            pltpu.sync_copy(agg.at[pl.ds(t * _RC, _RC)],
                            out_hbm.at[c, pl.ds(t * _RC, _RC)])


def _finish_body(hs_ref, agg0_ref, agg1_ref, batch_ref, eps_ref, wo_ref,
                 bo_ref, wxn_ref, hsn_ref, pooled_ref):
    i = pl.program_id(0)
    t = (1.0 + eps_ref[0, 0]) * hs_ref[...] + agg0_ref[...] + agg1_ref[...]
    u = jnp.dot(t, wo_ref[...], preferred_element_type=jnp.float32)
    h = jnp.maximum(u + bo_ref[...], 0.0)
    hsn_ref[...] = jnp.dot(h, wxn_ref[...], preferred_element_type=jnp.float32)
    oh = (batch_ref[...] ==
          lax.broadcasted_iota(jnp.int32, (1, NG), 1)).astype(jnp.float32)
    contrib = lax.dot_general(oh, h, (((0,), (0,)), ((), ())),
                              preferred_element_type=jnp.float32)

    @pl.when(i == 0)
    def _():
        pooled_ref[...] = jnp.zeros_like(pooled_ref)

    pooled_ref[...] += contrib


def _finish(hs, agg0, agg1, batch2, eps, wo, bo2, wxn):
    blk = 1000
    grid = N // blk
    return pl.pallas_call(
        _finish_body,
        grid=(grid,),
        in_specs=[
            pl.BlockSpec((blk, H), lambda i: (i, 0)),
            pl.BlockSpec((blk, H), lambda i: (i, 0)),
            pl.BlockSpec((blk, H), lambda i: (i, 0)),
            pl.BlockSpec((blk, 1), lambda i: (i, 0)),
            pl.BlockSpec((1, 1), lambda i: (0, 0)),
            pl.BlockSpec((H, H), lambda i: (0, 0)),
            pl.BlockSpec((1, H), lambda i: (0, 0)),
            pl.BlockSpec((H, H), lambda i: (0, 0)),
        ],
        out_specs=[
            pl.BlockSpec((blk, H), lambda i: (i, 0)),
            pl.BlockSpec((NG, H), lambda i: (0, 0)),
        ],
        out_shape=[
            jax.ShapeDtypeStruct((N, H), jnp.float32),
            jax.ShapeDtypeStruct((NG, H), jnp.float32),
        ],
    )(hs, agg0, agg1, batch2, eps, wo, bo2, wxn)


def _head_body(p1_ref, p2_ref, p3_ref, batch_ref, w1_ref, b1_ref, w2_ref,
               b2_ref, o_ref):
    oh = (batch_ref[...] ==
          lax.broadcasted_iota(jnp.int32, (1, NG), 1)).astype(jnp.float32)
    ones = jnp.ones((N, 1), jnp.float32)
    cnt = lax.dot_general(oh, ones, (((0,), (0,)), ((), ())),
                          preferred_element_type=jnp.float32)
    denom = jnp.maximum(cnt, 1.0)
    hcat = jnp.concatenate(
        [p1_ref[...] / denom, p2_ref[...] / denom, p3_ref[...] / denom],
        axis=1)
    hl = jnp.maximum(
        jnp.dot(hcat, w1_ref[...], preferred_element_type=jnp.float32)
        + b1_ref[...], 0.0)
    logits = jnp.dot(hl, w2_ref[...], preferred_element_type=jnp.float32) \
        + b2_ref[...]
    m = jnp.max(logits, axis=1, keepdims=True)
    lse = jnp.log(jnp.sum(jnp.exp(logits - m), axis=1, keepdims=True)) + m
    o_ref[...] = logits - lse


def _head(p1, p2, p3, batch2, w1, b12, w2, b22):
    return pl.pallas_call(
        _head_body,
        out_shape=jax.ShapeDtypeStruct((NG, OUT), jnp.float32),
    )(p1, p2, p3, batch2, w1, b12, w2, b22)


def kernel(x, edge_index, edge_attr, batch, Wx1, We1, Wo1, bo1, eps1, Wx2,
           We2, Wo2, bo2, eps2, Wx3, We3, Wo3, bo3, eps3, lin1_W, lin1_b,
           lin2_W, lin2_b):
    batch2 = batch.reshape(N, 1)
    eye = jnp.eye(H, dtype=jnp.float32)

    hs = _mm(x, Wx1, 1000)
    pooled = []
    layers = [
        (We1, Wo1, bo1, eps1, Wx2),
        (We2, Wo2, bo2, eps2, Wx3),
        (We3, Wo3, bo3, eps3, eye),
    ]
    for we, wo, bo, eps, wxn in layers:
        e = _mm(edge_attr, we, 4000)
        aggp = _sc_edge(hs, e, edge_index)
        hs, p = _finish(hs, aggp[0], aggp[1], batch2, eps.reshape(1, 1), wo,
                        bo.reshape(1, H), wxn)
        pooled.append(p)

    return _head(pooled[0], pooled[1], pooled[2], batch2, lin1_W,
                 lin1_b.reshape(1, 3 * H), lin2_W, lin2_b.reshape(1, OUT))
